# baseline (speedup 1.0000x reference)
import functools

import jax
import jax.numpy as jnp
from jax import lax
from jax.experimental import pallas as pl
from jax.experimental.pallas import tpu as pltpu
from jax.experimental.pallas import tpu_sc as plsc

BATCH = 4096
HIST = 200
DIM = 64
NW = 32
ROWS_W = BATCH // NW
CHUNK = 40
NCHUNK = ROWS_W * 5
NBUF = 4
NGROUP = NCHUNK // NBUF

_mesh = plsc.VectorSubcoreMesh(core_axis_name="c", subcore_axis_name="s")


def _body(table_hbm, idx_hbm, out_hbm, idx_v,
          b0, b1, b2, b3, g0, g1, g2, g3, s0, s1, s2, s3):
    bufs = (b0, b1, b2, b3)
    gsem = (g0, g1, g2, g3)
    ssem = (s0, s1, s2, s3)
    wid = lax.axis_index("s") * 2 + lax.axis_index("c")
    row0 = wid * ROWS_W

    pltpu.sync_copy(idx_hbm.at[pl.ds(row0, ROWS_W)], idx_v)

    def chunk_coords(j):
        return j // 5, (j % 5) * CHUNK

    def gather_start(j, b):
        r, h0 = chunk_coords(j)
        pltpu.async_copy(
            table_hbm.at[idx_v.at[r, pl.ds(h0, CHUNK)]], bufs[b], gsem[b])

    def gather_wait(j, b):
        r, h0 = chunk_coords(j)
        pltpu.make_async_copy(
            table_hbm.at[idx_v.at[r, pl.ds(h0, CHUNK)]], bufs[b], gsem[b]).wait()

    def store_start(j, b):
        r, h0 = chunk_coords(j)
        pltpu.async_copy(bufs[b].at[:, pl.ds(0, DIM)],
                         out_hbm.at[row0 + r, pl.ds(h0, CHUNK)], ssem[b])

    def store_wait(j, b):
        r, h0 = chunk_coords(j)
        pltpu.make_async_copy(bufs[b].at[:, pl.ds(0, DIM)],
                              out_hbm.at[row0 + r, pl.ds(h0, CHUNK)], ssem[b]).wait()

    for b in range(NBUF):
        gather_start(b, b)

    def group(g, carry):
        for b in range(NBUF):
            j = g * NBUF + b
            gather_wait(j, b)
            store_start(j, b)
            store_wait(j, b)
            gather_start(j + NBUF, b)
        return carry

    lax.fori_loop(0, NGROUP - 1, group, 0)

    for b in range(NBUF):
        j = (NGROUP - 1) * NBUF + b
        gather_wait(j, b)
        store_start(j, b)
    for b in range(NBUF):
        j = (NGROUP - 1) * NBUF + b
        store_wait(j, b)


_call = functools.partial(
    pl.kernel,
    mesh=_mesh,
    compiler_params=pltpu.CompilerParams(use_tc_tiling_on_sc=False),
    out_type=jax.ShapeDtypeStruct((BATCH, HIST, DIM), jnp.float32),
    scratch_types=(
        [pltpu.VMEM((ROWS_W, HIST), jnp.int32)]
        + [pltpu.VMEM((CHUNK, 2 * DIM), jnp.float32)] * NBUF
        + [pltpu.SemaphoreType.DMA] * (2 * NBUF)
    ),
)(_body)


def kernel(token_ids, embedding):
    table128 = jnp.pad(embedding, ((0, 0), (0, DIM)))
    return _call(table128, token_ids.astype(jnp.int32))


# speedup vs baseline: 1.2488x; 1.2488x over previous
import functools

import jax
import jax.numpy as jnp
from jax import lax
from jax.experimental import pallas as pl
from jax.experimental.pallas import tpu as pltpu
from jax.experimental.pallas import tpu_sc as plsc

BATCH = 4096
HIST = 200
DIM = 64
NW = 32
ROWS_W = BATCH // NW
PER_W = ROWS_W * HIST
CHUNK = 40
NCHUNK = PER_W // CHUNK
NBUF = 4
NGROUP = NCHUNK // NBUF

_mesh = plsc.VectorSubcoreMesh(core_axis_name="c", subcore_axis_name="s")


def _body(table_hbm, idx_hbm, out_hbm, idx_v,
          b0, b1, b2, b3, g0, g1, g2, g3, s0, s1, s2, s3):
    bufs = (b0, b1, b2, b3)
    gsem = (g0, g1, g2, g3)
    ssem = (s0, s1, s2, s3)
    wid = lax.axis_index("s") * 2 + lax.axis_index("c")

    pltpu.sync_copy(idx_hbm.at[pl.ds(wid * PER_W, PER_W)], idx_v)

    def gather_start(j, b):
        pltpu.async_copy(
            table_hbm.at[idx_v.at[pl.ds(j * CHUNK, CHUNK)]], bufs[b], gsem[b])

    def gather_wait(j, b):
        pltpu.make_async_copy(
            table_hbm.at[idx_v.at[pl.ds(j * CHUNK, CHUNK)]], bufs[b],
            gsem[b]).wait()

    def out_dst(j):
        return out_hbm.at[pl.ds(wid * PER_W + j * CHUNK, CHUNK)]

    def store_start(j, b):
        pltpu.async_copy(bufs[b], out_dst(j), ssem[b])

    def store_wait(j, b):
        pltpu.make_async_copy(bufs[b], out_dst(j), ssem[b]).wait()

    for b in range(NBUF):
        gather_start(b, b)

    def group(g, carry):
        for b in range(NBUF):
            j = g * NBUF + b
            gather_wait(j, b)
            store_start(j, b)
            store_wait(j, b)
            gather_start(j + NBUF, b)
        return carry

    lax.fori_loop(0, NGROUP - 1, group, 0)

    for b in range(NBUF):
        j = (NGROUP - 1) * NBUF + b
        gather_wait(j, b)
        store_start(j, b)
    for b in range(NBUF):
        j = (NGROUP - 1) * NBUF + b
        store_wait(j, b)


_call = functools.partial(
    pl.kernel,
    mesh=_mesh,
    out_type=jax.ShapeDtypeStruct((BATCH * HIST, 2 * DIM), jnp.float32),
    scratch_types=(
        [pltpu.VMEM((PER_W,), jnp.int32)]
        + [pltpu.VMEM((CHUNK, 2 * DIM), jnp.float32)] * NBUF
        + [pltpu.SemaphoreType.DMA] * (2 * NBUF)
    ),
)(_body)


def kernel(token_ids, embedding):
    table128 = jnp.pad(embedding, ((0, 0), (0, DIM)))
    out5 = _call(table128, token_ids.reshape(-1).astype(jnp.int32))
    return out5.reshape(BATCH, HIST, 2 * DIM)[:, :, :DIM]


# speedup vs baseline: 1.2616x; 1.0102x over previous
import functools

import jax
import jax.numpy as jnp
from jax import lax
from jax.experimental import pallas as pl
from jax.experimental.pallas import tpu as pltpu
from jax.experimental.pallas import tpu_sc as plsc

BATCH = 4096
HIST = 200
DIM = 64
NW = 32
ROWS_W = BATCH // NW
PER_W = ROWS_W * HIST
CHUNK = 128
NCHUNK = PER_W // CHUNK
NBUF = 4
NGROUP = NCHUNK // NBUF

_mesh = plsc.VectorSubcoreMesh(core_axis_name="c", subcore_axis_name="s")


def _body(table_hbm, idx_hbm, out_hbm, idx_v,
          b0, b1, b2, b3, g0, g1, g2, g3, s0, s1, s2, s3):
    bufs = (b0, b1, b2, b3)
    gsem = (g0, g1, g2, g3)
    ssem = (s0, s1, s2, s3)
    wid = lax.axis_index("s") * 2 + lax.axis_index("c")

    pltpu.sync_copy(idx_hbm.at[pl.ds(wid * PER_W, PER_W)], idx_v)

    def gather_start(j, b):
        pltpu.async_copy(
            table_hbm.at[idx_v.at[pl.ds(j * CHUNK, CHUNK)]], bufs[b], gsem[b])

    def gather_wait(j, b):
        pltpu.make_async_copy(
            table_hbm.at[idx_v.at[pl.ds(j * CHUNK, CHUNK)]], bufs[b],
            gsem[b]).wait()

    def out_dst(j):
        return out_hbm.at[pl.ds(wid * PER_W + j * CHUNK, CHUNK)]

    def store_start(j, b):
        pltpu.async_copy(bufs[b], out_dst(j), ssem[b])

    def store_wait(j, b):
        pltpu.make_async_copy(bufs[b], out_dst(j), ssem[b]).wait()

    for b in range(NBUF):
        gather_start(b, b)

    def group(g, carry):
        for b in range(NBUF):
            j = g * NBUF + b
            gather_wait(j, b)
            store_start(j, b)
            store_wait(j, b)
            gather_start(j + NBUF, b)
        return carry

    lax.fori_loop(0, NGROUP - 1, group, 0)

    for b in range(NBUF):
        j = (NGROUP - 1) * NBUF + b
        gather_wait(j, b)
        store_start(j, b)
    for b in range(NBUF):
        j = (NGROUP - 1) * NBUF + b
        store_wait(j, b)


_call = functools.partial(
    pl.kernel,
    mesh=_mesh,
    out_type=jax.ShapeDtypeStruct((BATCH * HIST, 2 * DIM), jnp.float32),
    scratch_types=(
        [pltpu.VMEM((PER_W,), jnp.int32)]
        + [pltpu.VMEM((CHUNK, 2 * DIM), jnp.float32)] * NBUF
        + [pltpu.SemaphoreType.DMA] * (2 * NBUF)
    ),
)(_body)


def kernel(token_ids, embedding):
    table128 = jnp.pad(embedding, ((0, 0), (0, DIM)))
    out5 = _call(table128, token_ids.reshape(-1).astype(jnp.int32))
    return out5.reshape(BATCH, HIST, 2 * DIM)[:, :, :DIM]


# speedup vs baseline: 1.2627x; 1.0009x over previous
import functools

import jax
import jax.numpy as jnp
from jax import lax
from jax.experimental import pallas as pl
from jax.experimental.pallas import tpu as pltpu
from jax.experimental.pallas import tpu_sc as plsc

BATCH = 4096
HIST = 200
DIM = 64
NW = 32
ROWS_W = BATCH // NW
PER_W = ROWS_W * HIST
CHUNK = 128
NCHUNK = PER_W // CHUNK
NBUF = 5
NGROUP = NCHUNK // NBUF

_mesh = plsc.VectorSubcoreMesh(core_axis_name="c", subcore_axis_name="s")


def _body(table_hbm, idx_hbm, out_hbm, idx_v,
          b0, b1, b2, b3, b4,
          g0, g1, g2, g3, g4, s0, s1, s2, s3, s4):
    bufs = (b0, b1, b2, b3, b4)
    gsem = (g0, g1, g2, g3, g4)
    ssem = (s0, s1, s2, s3, s4)
    wid = lax.axis_index("s") * 2 + lax.axis_index("c")

    pltpu.sync_copy(idx_hbm.at[pl.ds(wid * PER_W, PER_W)], idx_v)

    def gather_start(j, b):
        pltpu.async_copy(
            table_hbm.at[idx_v.at[pl.ds(j * CHUNK, CHUNK)]], bufs[b], gsem[b])

    def gather_wait(j, b):
        pltpu.make_async_copy(
            table_hbm.at[idx_v.at[pl.ds(j * CHUNK, CHUNK)]], bufs[b],
            gsem[b]).wait()

    def out_dst(j):
        return out_hbm.at[pl.ds(wid * PER_W + j * CHUNK, CHUNK)]

    def store_start(j, b):
        pltpu.async_copy(bufs[b], out_dst(j), ssem[b])

    def store_wait(j, b):
        pltpu.make_async_copy(bufs[b], out_dst(j), ssem[b]).wait()

    for b in range(NBUF):
        gather_start(b, b)

    def group(g, carry):
        for b in range(NBUF):
            j = g * NBUF + b
            gather_wait(j, b)
            store_start(j, b)
            store_wait(j, b)
            gather_start(j + NBUF, b)
        return carry

    lax.fori_loop(0, NGROUP - 1, group, 0)

    for b in range(NBUF):
        j = (NGROUP - 1) * NBUF + b
        gather_wait(j, b)
        store_start(j, b)
    for b in range(NBUF):
        j = (NGROUP - 1) * NBUF + b
        store_wait(j, b)


_call = functools.partial(
    pl.kernel,
    mesh=_mesh,
    out_type=jax.ShapeDtypeStruct((BATCH * HIST, 2 * DIM), jnp.float32),
    scratch_types=(
        [pltpu.VMEM((PER_W,), jnp.int32)]
        + [pltpu.VMEM((CHUNK, 2 * DIM), jnp.float32)] * NBUF
        + [pltpu.SemaphoreType.DMA] * (2 * NBUF)
    ),
)(_body)


def kernel(token_ids, embedding):
    table128 = jnp.pad(embedding, ((0, 0), (0, DIM)))
    out5 = _call(table128, token_ids.reshape(-1).astype(jnp.int32))
    return out5.reshape(BATCH, HIST, 2 * DIM)[:, :, :DIM]
